# Initial kernel scaffold; baseline (speedup 1.0000x reference)
import jax
import jax.numpy as jnp
from jax.experimental import pallas as pl


def kernel(x, wp0, b0, mask0, g0, bt0, wp1, b1, mask1, g1, bt1, wp2, b2, mask2, g2, bt2, wp3, b3, mask3):
    raise NotImplementedError("write your pallas kernel here")


# speedup vs baseline: 1.0491x; 1.0491x over previous
import jax
import jax.numpy as jnp
from jax.experimental import pallas as pl
from jax.experimental.pallas import tpu as pltpu

_EPS = 1e-5


def _slabs9(x_ref, mask_ref, xp_ref):
    M, Cin = x_ref.shape
    PAD = (xp_ref.shape[0] - M) // 2
    W = PAD - 1
    zeros = jnp.zeros((PAD, Cin), xp_ref.dtype)
    xp_ref[0:PAD, :] = zeros
    xp_ref[PAD + M:PAD + 2 * PAD + M, :] = zeros
    xp_ref[PAD:PAD + M, :] = x_ref[...]
    mask = mask_ref[...]
    out = []
    for dy in range(3):
        for dx in range(3):
            k = dy * 3 + dx
            off = PAD + (dy - 1) * W + (dx - 1)
            v = xp_ref[off:off + M, :]
            if k != 4:
                v = v * mask[:, k:k + 1]
            out.append(v)
    return out


def _phase_pair_dots(patch_ref, w_ref, Cin, C):
    ys = []
    for ry in range(2):
        for rx in range(2):
            p = 2 * ry + rx
            acc = None
            for a in range(2):
                k0 = (ry + a) * 3 + rx
                lhs = patch_ref[:, k0 * Cin:(k0 + 2) * Cin]
                rhs = w_ref[k0 * Cin:(k0 + 2) * Cin, p * C:(p + 1) * C]
                d = jnp.dot(lhs, rhs, preferred_element_type=jnp.float32)
                acc = d if acc is None else acc + d
            ys.append(acc)
    return ys


def _bn_tanh_store(ys, g_ref, bt_ref, o_ref):
    C = ys[0].shape[1]
    cnt = float(4 * ys[0].shape[0])
    s = ys[0].sum(axis=0, keepdims=True)
    for y in ys[1:]:
        s = s + y.sum(axis=0, keepdims=True)
    mean = s / cnt
    sq = None
    ds = []
    for y in ys:
        d = y - mean
        ds.append(d)
        t = (d * d).sum(axis=0, keepdims=True)
        sq = t if sq is None else sq + t
    var = sq / cnt
    scale = g_ref[...] * jax.lax.rsqrt(var + _EPS)
    bt = bt_ref[...]
    for p in range(4):
        o_ref[:, p * C:(p + 1) * C] = jnp.tanh(ds[p] * scale + bt)


def _bn_layer_pairdot_kernel(x_ref, w_ref, b_ref, g_ref, bt_ref, mask_ref,
                             o_ref, xp_ref, patch_ref):
    Cin = x_ref.shape[1]
    C = b_ref.shape[1]
    slabs = _slabs9(x_ref, mask_ref, xp_ref)
    for k in range(9):
        patch_ref[:, k * Cin:(k + 1) * Cin] = slabs[k]
    ys = _phase_pair_dots(patch_ref, w_ref, Cin, C)
    _bn_tanh_store(ys, g_ref, bt_ref, o_ref)


def _bn_layer_bigdot_kernel(x_ref, w_ref, b_ref, g_ref, bt_ref, mask_ref,
                            o_ref, xp_ref, patch_ref):
    Cin = x_ref.shape[1]
    C = b_ref.shape[1]
    slabs = _slabs9(x_ref, mask_ref, xp_ref)
    for k in range(9):
        patch_ref[:, k * Cin:(k + 1) * Cin] = slabs[k]
    y = jnp.dot(patch_ref[...], w_ref[...],
                preferred_element_type=jnp.float32)
    ys = [y[:, p * C:(p + 1) * C] for p in range(4)]
    _bn_tanh_store(ys, g_ref, bt_ref, o_ref)


def _sig_layer_kernel(x_ref, w_ref, b_ref, mask_ref, o_ref, xp_ref,
                      patch_ref):
    Cin = x_ref.shape[1]
    slabs = _slabs9(x_ref, mask_ref, xp_ref)
    for k in range(9):
        patch_ref[:, k * Cin:(k + 1) * Cin] = slabs[k]
    y = jnp.dot(patch_ref[...], w_ref[...],
                preferred_element_type=jnp.float32)
    C = b_ref.shape[1]
    b4 = jnp.concatenate([b_ref[...]] * 4, axis=-1)
    o_ref[...] = pl.reciprocal(1.0 + jnp.exp(-(y + b4)), approx=True)


def _whole(shape):
    return pl.BlockSpec(shape, lambda *_: (0,) * len(shape))


def _bn_layer(xf, wp, b, g, bt, mask, W, pairdot):
    M, Cin = xf.shape
    C4 = wp.shape[1]
    PAD = W + 1
    body = _bn_layer_pairdot_kernel if pairdot else _bn_layer_bigdot_kernel
    return pl.pallas_call(
        body,
        grid=(1,),
        in_specs=[_whole((M, Cin)), _whole(wp.shape), _whole(b.shape),
                  _whole(g.shape), _whole(bt.shape), _whole(mask.shape)],
        out_specs=_whole((M, C4)),
        out_shape=jax.ShapeDtypeStruct((M, C4), jnp.float32),
        scratch_shapes=[pltpu.VMEM((M + 2 * PAD, Cin), jnp.float32),
                        pltpu.VMEM((M, 9 * Cin), jnp.float32)],
        compiler_params=pltpu.CompilerParams(
            dimension_semantics=("arbitrary",)),
    )(xf, wp, b, g, bt, mask)


def _sig_layer(xf, wp, b, mask, W):
    M, Cin = xf.shape
    C4 = wp.shape[1]
    PAD = W + 1
    MH = M // 2
    return pl.pallas_call(
        _sig_layer_kernel,
        grid=(2,),
        in_specs=[pl.BlockSpec((MH, Cin), lambda i: (i, 0)),
                  _whole(wp.shape), _whole(b.shape),
                  pl.BlockSpec((MH, 9), lambda i: (i, 0))],
        out_specs=pl.BlockSpec((MH, C4), lambda i: (i, 0)),
        out_shape=jax.ShapeDtypeStruct((M, C4), jnp.float32),
        scratch_shapes=[pltpu.VMEM((MH + 2 * PAD, Cin), jnp.float32),
                        pltpu.VMEM((MH, 9 * Cin), jnp.float32)],
        compiler_params=pltpu.CompilerParams(
            dimension_semantics=("parallel",)),
    )(xf, wp, b, mask)


def kernel(x, wp0, b0, mask0, g0, bt0, wp1, b1, mask1, g1, bt1,
           wp2, b2, mask2, g2, bt2, wp3, b3, mask3):
    layers = [
        (wp0, b0, g0, bt0, mask0, True),
        (wp1, b1, g1, bt1, mask1, True),
        (wp2, b2, g2, bt2, mask2, False),
        (wp3, b3, None, None, mask3, False),
    ]
    h = jnp.transpose(x, (0, 2, 3, 1)).astype(jnp.float32)
    for wp, b, g, bt, mask, pairdot in layers:
        B, H, W, Cin = h.shape
        C = wp.shape[1] // 4
        xf = h.reshape(B * H * W, Cin)
        if g is not None:
            y = _bn_layer(xf, wp, b, g, bt, mask, W, pairdot)
        else:
            y = _sig_layer(xf, wp, b, mask, W)
        y = y.reshape(B, H, W, 2, 2, C)
        h = jnp.transpose(y, (0, 1, 3, 2, 4, 5)).reshape(B, 2 * H, 2 * W, C)
    return jnp.transpose(h, (0, 3, 1, 2))
